# Initial kernel scaffold; baseline (speedup 1.0000x reference)
import functools

import jax
import jax.numpy as jnp
from jax import lax
from jax.experimental import pallas as pl
from jax.experimental.pallas import tpu as pltpu, tpu_sc as plsc

_B = 16384
_N_LOC = 128
_N_GROUPS = 16
_L = 16

_info = plsc.get_sparse_core_info()
_NC, _NS = _info.num_cores, _info.num_subcores
_NW = _NC * _NS
_BPW = _B // _NW


@functools.partial(
    pl.kernel,
    out_type=jax.ShapeDtypeStruct((_B,), jnp.float32),
    mesh=plsc.VectorSubcoreMesh(core_axis_name="c", subcore_axis_name="s"),
    scratch_types=[
        pltpu.VMEM((_N_GROUPS,), jnp.float32),
        pltpu.VMEM((_N_LOC,), jnp.int32),
        pltpu.VMEM((_N_LOC,), jnp.float32),
        pltpu.VMEM((_BPW,), jnp.int32),
        pltpu.VMEM((_BPW,), jnp.float32),
    ],
)
def _lookup(params_hbm, loc_hbm, l2g_hbm, out_hbm,
            params_v, l2g_v, table_v, loc_v, out_v):
    wid = lax.axis_index("s") * _NC + lax.axis_index("c")
    base = wid * _BPW
    pltpu.sync_copy(params_hbm, params_v)
    pltpu.sync_copy(l2g_hbm, l2g_v)
    pltpu.sync_copy(loc_hbm.at[pl.ds(base, _BPW)], loc_v)
    for i in range(_N_LOC // _L):
        g = l2g_v[pl.ds(i * _L, _L)]
        table_v[pl.ds(i * _L, _L)] = plsc.load_gather(params_v, [g])
    for j in range(_BPW // _L):
        idx = loc_v[pl.ds(j * _L, _L)]
        out_v[pl.ds(j * _L, _L)] = plsc.load_gather(table_v, [idx])
    pltpu.sync_copy(out_v, out_hbm.at[pl.ds(base, _BPW)])


def kernel(params, location, loc_to_group):
    out = _lookup(params.astype(jnp.float32),
                  location.astype(jnp.int32),
                  loc_to_group.astype(jnp.int32))
    return out.reshape(-1, 1)


# speedup vs baseline: 6.5396x; 6.5396x over previous
import functools

import jax
import jax.numpy as jnp
from jax import lax
from jax.experimental import pallas as pl
from jax.experimental.pallas import tpu as pltpu, tpu_sc as plsc

_B = 16384
_N_LOC = 128
_N_GROUPS = 16
_L = 16

_info = plsc.get_sparse_core_info()
_NC, _NS = _info.num_cores, _info.num_subcores
_NW = _NC * _NS
_BPW = _B // _NW


@functools.partial(
    pl.kernel,
    out_type=jax.ShapeDtypeStruct((_B,), jnp.float32),
    mesh=plsc.VectorSubcoreMesh(core_axis_name="c", subcore_axis_name="s"),
    compiler_params=pltpu.CompilerParams(needs_layout_passes=False),
    scratch_types=[
        pltpu.VMEM((_N_GROUPS,), jnp.float32),
        pltpu.VMEM((_N_LOC,), jnp.int32),
        pltpu.VMEM((_N_LOC,), jnp.float32),
        pltpu.VMEM((_BPW,), jnp.int32),
        pltpu.VMEM((_BPW,), jnp.float32),
    ],
)
def _lookup(params_hbm, loc_hbm, l2g_hbm, out_hbm,
            params_v, l2g_v, table_v, loc_v, out_v):
    wid = lax.axis_index("s") * _NC + lax.axis_index("c")
    base = wid * _BPW
    pltpu.sync_copy(params_hbm, params_v)
    pltpu.sync_copy(l2g_hbm, l2g_v)
    pltpu.sync_copy(loc_hbm.at[pl.ds(base, _BPW)], loc_v)
    for i in range(_N_LOC // _L):
        g = l2g_v[pl.ds(i * _L, _L)]
        table_v[pl.ds(i * _L, _L)] = plsc.load_gather(params_v, [g])
    for j in range(_BPW // _L):
        idx = loc_v[pl.ds(j * _L, _L)]
        out_v[pl.ds(j * _L, _L)] = plsc.load_gather(table_v, [idx])
    pltpu.sync_copy(out_v, out_hbm.at[pl.ds(base, _BPW)])


def kernel(params, location, loc_to_group):
    out = _lookup(params.astype(jnp.float32),
                  location.astype(jnp.int32),
                  loc_to_group.astype(jnp.int32))
    return out.reshape(-1, 1)
